# baseline (speedup 1.0000x reference)
import functools

import jax
import jax.numpy as jnp
from jax import lax
from jax.experimental import pallas as pl
from jax.experimental.pallas import tpu as pltpu
from jax.experimental.pallas import tpu_sc as plsc

_D = 32
_L = 16
_NC = 2
_NS = 16
_NW = _NC * _NS

_B = 16384
_T = 50
_TP = 56
_FAT = 128

_CB = 4
_CR = _CB * _T
_CRP = 208
_CF = _CB * _TP


def _perm(v, idx):
    return lax.gather(
        v, idx[:, None],
        dimension_numbers=lax.GatherDimensionNumbers(
            offset_dims=(), collapsed_slice_dims=(0,), start_index_map=(0,)),
        slice_sizes=(1,),
        mode=lax.GatherScatterMode.PROMISE_IN_BOUNDS)


def _newton_rsqrt(x):
    i = lax.bitcast_convert_type(x, jnp.int32)
    i = jnp.int32(0x5F3759DF) - lax.shift_right_arithmetic(i, 1)
    y = lax.bitcast_convert_type(i, jnp.float32)
    for _ in range(3):
        y = y * (jnp.float32(1.5) - jnp.float32(0.5) * x * y * y)
    return y


def _normalize_group(rows_ref, fat_ref, idx_ref, r0, n):
    lane = lax.iota(jnp.int32, _L)
    perms = [(lane ^ st).astype(jnp.int32) for st in (1, 2, 4, 8)]

    quarter = lax.shift_right_logical(idx_ref[pl.ds(r0, _L)], 3) & jnp.int32(3)
    a = []
    b = []
    for r in range(n):
        off = quarter[r] * _D
        a.append(rows_ref[r0 + r, pl.ds(off, _L)])
        b.append(rows_ref[r0 + r, pl.ds(off + _L, _L)])
    S = jnp.zeros((_L,), jnp.float32)
    for r in range(n):
        s = a[r] * a[r] + b[r] * b[r]
        for p in perms:
            s = s + _perm(s, p)
        S = jnp.where(lane == r, s, S)
    scale = _newton_rsqrt(S * jnp.float32(1.0 / _D) + jnp.float32(1e-8))
    for r in range(n):
        sc = _perm(scale, jnp.full((_L,), r, jnp.int32))
        cr = r0 + r
        fr = cr + 6 * (cr // _T)
        fat_ref[fr, pl.ds(0, _L)] = a[r] * sc
        fat_ref[fr, pl.ds(_L, _L)] = b[r] * sc


def _make_sc_kernel():
    n_rows = _B * _T
    rows_per_worker = n_rows // _NW
    batches_per_worker = _B // _NW
    n_chunks = batches_per_worker // _CB
    fat_rows = _B * _TP
    n_groups = _CR // _L
    tail = _CR - n_groups * _L

    mesh = plsc.VectorSubcoreMesh(core_axis_name="c", subcore_axis_name="s")

    @functools.partial(
        pl.kernel,
        out_type=jax.ShapeDtypeStruct((fat_rows, _FAT), jnp.float32),
        mesh=mesh,
        scratch_types=[
            pltpu.VMEM((_CRP,), jnp.int32),
            pltpu.VMEM((_CRP,), jnp.int32),
            pltpu.VMEM((_CRP,), jnp.int32),
            pltpu.VMEM((_CRP,), jnp.int32),
            pltpu.VMEM((_CR, _FAT), jnp.float32),
            pltpu.VMEM((_CR, _FAT), jnp.float32),
            pltpu.VMEM((_CF, _FAT), jnp.float32),
            pltpu.SemaphoreType.DMA,
            pltpu.SemaphoreType.DMA,
            pltpu.SemaphoreType.DMA,
        ],
        compiler_params=pltpu.CompilerParams(use_tc_tiling_on_sc=True),
    )
    def sc_kernel(idx_hbm, w_hbm, out_hbm,
                  idx0, idx1, fidx0, fidx1, rows0, rows1, fat,
                  sg0, sg1, so):
        wid = lax.axis_index("s") * _NC + lax.axis_index("c")
        idx_base = wid * rows_per_worker
        fat_base = wid * batches_per_worker * _TP

        idx_bufs = [idx0, idx1]
        fidx_bufs = [fidx0, fidx1]
        row_bufs = [rows0, rows1]
        g_sems = [sg0, sg1]

        def zero_body(i, carry):
            for c in range(_FAT // _L):
                fat[i, pl.ds(c * _L, _L)] = jnp.zeros((_L,), jnp.float32)
            return carry
        lax.fori_loop(0, _CF, zero_body, 0)

        def gather_slices(p):
            return (w_hbm.at[fidx_bufs[p].at[pl.ds(0, _CR)]], row_bufs[p])

        def gather_start(g, p):
            pltpu.sync_copy(idx_hbm.at[pl.ds(idx_base + g * _CR, _CR)],
                            idx_bufs[p].at[pl.ds(0, _CR)])
            for k in range(_CRP // _L):
                iv = idx_bufs[p][pl.ds(k * _L, _L)]
                fidx_bufs[p][pl.ds(k * _L, _L)] = (
                    lax.shift_right_logical(iv, 5) * jnp.int32(8)
                    + (iv & jnp.int32(7)))
            src, dst = gather_slices(p)
            pltpu.async_copy(src, dst, g_sems[p])

        def gather_wait(p):
            src, dst = gather_slices(p)
            pltpu.make_async_copy(src, dst, g_sems[p]).wait()

        def out_start(g):
            pltpu.async_copy(
                fat, out_hbm.at[pl.ds(fat_base + g * _CF, _CF)], so)

        def out_wait(g):
            pltpu.make_async_copy(
                fat, out_hbm.at[pl.ds(fat_base + g * _CF, _CF)], so).wait()

        gather_start(0, 0)

        def pair_body(gp, carry):
            for sub in (0, 1):
                g = 2 * gp + sub

                @pl.when(g + 1 < n_chunks)
                def _():
                    gather_start(g + 1, 1 - sub)

                gather_wait(sub)

                @pl.when(g >= 1)
                def _():
                    out_wait(g - 1)

                def group_body(i, carry2):
                    _normalize_group(row_bufs[sub], fat, idx_bufs[sub],
                                     i * _L, _L)
                    return carry2
                lax.fori_loop(0, n_groups, group_body, 0)
                if tail:
                    _normalize_group(row_bufs[sub], fat, idx_bufs[sub],
                                     n_groups * _L, tail)
                out_start(g)
            return carry

        lax.fori_loop(0, n_chunks // 2, pair_body, 0)
        out_wait(n_chunks - 1)

    return sc_kernel


def kernel(x, weight):
    idx = x.reshape(-1).astype(jnp.int32)
    tbl = weight.reshape(-1, 4, 8, _D).transpose(0, 2, 1, 3)
    tbl = tbl.reshape(-1, _FAT)
    out_fat = _make_sc_kernel()(idx, tbl)
    return out_fat.reshape(_B, _TP, _FAT)[:, :_T, :_D]


# speedup vs baseline: 1.2816x; 1.2816x over previous
import functools

import jax
import jax.numpy as jnp
from jax import lax
from jax.experimental import pallas as pl
from jax.experimental.pallas import tpu as pltpu
from jax.experimental.pallas import tpu_sc as plsc

_D = 32
_L = 16
_NC = 2
_NS = 16
_NW = _NC * _NS

_B = 16384
_T = 50
_TP = 56
_FAT = 128

_CB = 8
_CR = _CB * _T
_CF = _CB * _TP


def _perm(v, idx):
    return lax.gather(
        v, idx[:, None],
        dimension_numbers=lax.GatherDimensionNumbers(
            offset_dims=(), collapsed_slice_dims=(0,), start_index_map=(0,)),
        slice_sizes=(1,),
        mode=lax.GatherScatterMode.PROMISE_IN_BOUNDS)


def _newton_rsqrt(x):
    i = lax.bitcast_convert_type(x, jnp.int32)
    i = jnp.int32(0x5F3759DF) - lax.shift_right_arithmetic(i, 1)
    y = lax.bitcast_convert_type(i, jnp.float32)
    for _ in range(3):
        y = y * (jnp.float32(1.5) - jnp.float32(0.5) * x * y * y)
    return y


def _normalize_chunk_to_fat(rows_ref, fat_ref):
    lane = lax.iota(jnp.int32, _L)
    perms = [(lane ^ st).astype(jnp.int32) for st in (1, 2, 4, 8)]

    def group_body(i, carry):
        r0 = i * _L
        a = [rows_ref[r0 + r, pl.ds(0, _L)] for r in range(_L)]
        b = [rows_ref[r0 + r, pl.ds(_L, _L)] for r in range(_L)]
        S = jnp.zeros((_L,), jnp.float32)
        for r in range(_L):
            s = a[r] * a[r] + b[r] * b[r]
            for p in perms:
                s = s + _perm(s, p)
            S = jnp.where(lane == r, s, S)
        scale = _newton_rsqrt(S * jnp.float32(1.0 / _D) + jnp.float32(1e-8))
        for r in range(_L):
            sc = _perm(scale, jnp.full((_L,), r, jnp.int32))
            cr = r0 + r
            fr = cr + 6 * (cr // _T)
            fat_ref[fr, pl.ds(0, _L)] = a[r] * sc
            fat_ref[fr, pl.ds(_L, _L)] = b[r] * sc
        return carry

    lax.fori_loop(0, _CR // _L, group_body, 0)


def _make_sc_kernel():
    n_rows = _B * _T
    rows_per_worker = n_rows // _NW
    batches_per_worker = _B // _NW
    n_chunks = batches_per_worker // _CB
    fat_rows = _B * _TP

    mesh = plsc.VectorSubcoreMesh(core_axis_name="c", subcore_axis_name="s")

    @functools.partial(
        pl.kernel,
        out_type=jax.ShapeDtypeStruct((fat_rows, _FAT), jnp.float32),
        mesh=mesh,
        scratch_types=[
            pltpu.VMEM((_CR,), jnp.int32),
            pltpu.VMEM((_CR,), jnp.int32),
            pltpu.VMEM((_CR, _D), jnp.float32),
            pltpu.VMEM((_CR, _D), jnp.float32),
            pltpu.VMEM((_CF, _FAT), jnp.float32),
            pltpu.SemaphoreType.DMA,
            pltpu.SemaphoreType.DMA,
            pltpu.SemaphoreType.DMA,
        ],
        compiler_params=pltpu.CompilerParams(use_tc_tiling_on_sc=False),
    )
    def sc_kernel(idx_hbm, w_hbm, out_hbm,
                  idx0, idx1, rows0, rows1, fat, sg0, sg1, so):
        wid = lax.axis_index("s") * _NC + lax.axis_index("c")
        idx_base = wid * rows_per_worker
        fat_base = wid * batches_per_worker * _TP

        idx_bufs = [idx0, idx1]
        row_bufs = [rows0, rows1]
        g_sems = [sg0, sg1]

        def zero_body(i, carry):
            for c in range(_FAT // _L):
                fat[i, pl.ds(c * _L, _L)] = jnp.zeros((_L,), jnp.float32)
            return carry
        lax.fori_loop(0, _CF, zero_body, 0)

        def gather_start(g, p):
            pltpu.sync_copy(idx_hbm.at[pl.ds(idx_base + g * _CR, _CR)],
                            idx_bufs[p])
            pltpu.async_copy(w_hbm.at[idx_bufs[p]], row_bufs[p], g_sems[p])

        def gather_wait(p):
            pltpu.make_async_copy(w_hbm.at[idx_bufs[p]], row_bufs[p],
                                  g_sems[p]).wait()

        def out_start(g):
            pltpu.async_copy(
                fat, out_hbm.at[pl.ds(fat_base + g * _CF, _CF)], so)

        def out_wait(g):
            pltpu.make_async_copy(
                fat, out_hbm.at[pl.ds(fat_base + g * _CF, _CF)], so).wait()

        gather_start(0, 0)

        def pair_body(gp, carry):
            for sub in (0, 1):
                g = 2 * gp + sub

                @pl.when(g + 1 < n_chunks)
                def _():
                    gather_start(g + 1, 1 - sub)

                gather_wait(sub)

                @pl.when(g >= 1)
                def _():
                    out_wait(g - 1)

                _normalize_chunk_to_fat(row_bufs[sub], fat)
                out_start(g)
            return carry

        lax.fori_loop(0, n_chunks // 2, pair_body, 0)
        out_wait(n_chunks - 1)

    return sc_kernel


def kernel(x, weight):
    idx = x.reshape(-1).astype(jnp.int32)
    out_fat = _make_sc_kernel()(idx, weight)
    return out_fat.reshape(_B, _TP, _FAT)[:, :_T, :_D]


# speedup vs baseline: 1.3228x; 1.0322x over previous
import functools

import jax
import jax.numpy as jnp
from jax import lax
from jax.experimental import pallas as pl
from jax.experimental.pallas import tpu as pltpu
from jax.experimental.pallas import tpu_sc as plsc

_D = 32
_L = 16
_NC = 2
_NS = 16
_NW = _NC * _NS

_B = 16384
_T = 50
_TP = 56
_FAT = 128

_CB = 4
_CR = _CB * _T
_CRP = 208
_CF = _CB * _TP


def _perm(v, idx):
    return lax.gather(
        v, idx[:, None],
        dimension_numbers=lax.GatherDimensionNumbers(
            offset_dims=(), collapsed_slice_dims=(0,), start_index_map=(0,)),
        slice_sizes=(1,),
        mode=lax.GatherScatterMode.PROMISE_IN_BOUNDS)


def _newton_rsqrt(x):
    i = lax.bitcast_convert_type(x, jnp.int32)
    i = jnp.int32(0x5F3759DF) - lax.shift_right_arithmetic(i, 1)
    y = lax.bitcast_convert_type(i, jnp.float32)
    for _ in range(3):
        y = y * (jnp.float32(1.5) - jnp.float32(0.5) * x * y * y)
    return y


def _normalize_group(rows_ref, fat_ref, idx_ref, r0, n):
    lane = lax.iota(jnp.int32, _L)
    perms = [(lane ^ st).astype(jnp.int32) for st in (1, 2, 4, 8)]

    quarter = idx_ref[pl.ds(r0, _L)] & jnp.int32(3)
    a = []
    b = []
    for r in range(n):
        off = quarter[r] * _D
        a.append(rows_ref[r0 + r, pl.ds(off, _L)])
        b.append(rows_ref[r0 + r, pl.ds(off + _L, _L)])
    S = jnp.zeros((_L,), jnp.float32)
    for r in range(n):
        s = a[r] * a[r] + b[r] * b[r]
        for p in perms:
            s = s + _perm(s, p)
        S = jnp.where(lane == r, s, S)
    scale = _newton_rsqrt(S * jnp.float32(1.0 / _D) + jnp.float32(1e-8))
    for r in range(n):
        sc = _perm(scale, jnp.full((_L,), r, jnp.int32))
        cr = r0 + r
        fr = cr + 6 * (cr // _T)
        fat_ref[fr, pl.ds(0, _L)] = a[r] * sc
        fat_ref[fr, pl.ds(_L, _L)] = b[r] * sc


def _make_sc_kernel():
    n_rows = _B * _T
    rows_per_worker = n_rows // _NW
    batches_per_worker = _B // _NW
    n_chunks = batches_per_worker // _CB
    fat_rows = _B * _TP
    n_groups = _CR // _L
    tail = _CR - n_groups * _L

    mesh = plsc.VectorSubcoreMesh(core_axis_name="c", subcore_axis_name="s")

    @functools.partial(
        pl.kernel,
        out_type=jax.ShapeDtypeStruct((fat_rows, _FAT), jnp.float32),
        mesh=mesh,
        scratch_types=[
            pltpu.VMEM((_CRP,), jnp.int32),
            pltpu.VMEM((_CRP,), jnp.int32),
            pltpu.VMEM((_CRP,), jnp.int32),
            pltpu.VMEM((_CRP,), jnp.int32),
            pltpu.VMEM((_CR, _FAT), jnp.float32),
            pltpu.VMEM((_CR, _FAT), jnp.float32),
            pltpu.VMEM((_CF, _FAT), jnp.float32),
            pltpu.SemaphoreType.DMA,
            pltpu.SemaphoreType.DMA,
            pltpu.SemaphoreType.DMA,
        ],
        compiler_params=pltpu.CompilerParams(use_tc_tiling_on_sc=True),
    )
    def sc_kernel(idx_hbm, w_hbm, out_hbm,
                  idx0, idx1, fidx0, fidx1, rows0, rows1, fat,
                  sg0, sg1, so):
        wid = lax.axis_index("s") * _NC + lax.axis_index("c")
        idx_base = wid * rows_per_worker
        fat_base = wid * batches_per_worker * _TP

        idx_bufs = [idx0, idx1]
        fidx_bufs = [fidx0, fidx1]
        row_bufs = [rows0, rows1]
        g_sems = [sg0, sg1]

        def zero_body(i, carry):
            for c in range(_FAT // _L):
                fat[i, pl.ds(c * _L, _L)] = jnp.zeros((_L,), jnp.float32)
            return carry
        lax.fori_loop(0, _CF, zero_body, 0)

        def gather_slices(p):
            return (w_hbm.at[fidx_bufs[p].at[pl.ds(0, _CR)]], row_bufs[p])

        def gather_start(g, p):
            pltpu.sync_copy(idx_hbm.at[pl.ds(idx_base + g * _CR, _CR)],
                            idx_bufs[p].at[pl.ds(0, _CR)])
            for k in range(_CRP // _L):
                fidx_bufs[p][pl.ds(k * _L, _L)] = lax.shift_right_logical(
                    idx_bufs[p][pl.ds(k * _L, _L)], 2)
            src, dst = gather_slices(p)
            pltpu.async_copy(src, dst, g_sems[p])

        def gather_wait(p):
            src, dst = gather_slices(p)
            pltpu.make_async_copy(src, dst, g_sems[p]).wait()

        def out_start(g):
            pltpu.async_copy(
                fat, out_hbm.at[pl.ds(fat_base + g * _CF, _CF)], so)

        def out_wait(g):
            pltpu.make_async_copy(
                fat, out_hbm.at[pl.ds(fat_base + g * _CF, _CF)], so).wait()

        gather_start(0, 0)

        def pair_body(gp, carry):
            for sub in (0, 1):
                g = 2 * gp + sub

                @pl.when(g + 1 < n_chunks)
                def _():
                    gather_start(g + 1, 1 - sub)

                gather_wait(sub)

                @pl.when(g >= 1)
                def _():
                    out_wait(g - 1)

                def group_body(i, carry2):
                    _normalize_group(row_bufs[sub], fat, idx_bufs[sub],
                                     i * _L, _L)
                    return carry2
                lax.fori_loop(0, n_groups, group_body, 0)
                if tail:
                    _normalize_group(row_bufs[sub], fat, idx_bufs[sub],
                                     n_groups * _L, tail)
                out_start(g)
            return carry

        lax.fori_loop(0, n_chunks // 2, pair_body, 0)
        out_wait(n_chunks - 1)

    return sc_kernel


_DT_ROWS = 256
_DT_Q = _DT_ROWS // 4


def _make_detile_kernel(n_table_rows):
    n_quads = n_table_rows // 4
    n_chunks = -(-n_table_rows // _DT_ROWS)
    last_r0 = n_table_rows - _DT_ROWS
    n_iter = -(-n_chunks // _NW)

    mesh = plsc.VectorSubcoreMesh(core_axis_name="c", subcore_axis_name="s")

    @functools.partial(
        pl.kernel,
        out_type=jax.ShapeDtypeStruct((n_quads, _FAT), jnp.float32),
        mesh=mesh,
        scratch_types=[
            pltpu.VMEM((_DT_ROWS // 8, 8, _D), jnp.float32),
            pltpu.VMEM((_DT_ROWS // 8, 8, _D), jnp.float32),
            pltpu.VMEM((_DT_Q, _FAT), jnp.float32),
            pltpu.VMEM((_DT_Q, _FAT), jnp.float32),
            pltpu.SemaphoreType.DMA,
            pltpu.SemaphoreType.DMA,
            pltpu.SemaphoreType.DMA,
            pltpu.SemaphoreType.DMA,
        ],
        compiler_params=pltpu.CompilerParams(use_tc_tiling_on_sc=True),
    )
    def detile(w_hbm, tbl_hbm, vin0, vin1, vout0, vout1, si0, si1, so0, so1):
        wid = lax.axis_index("s") * _NC + lax.axis_index("c")
        vins = [vin0, vin1]
        vouts = [vout0, vout1]
        isems = [si0, si1]
        osems = [so0, so1]

        def chunk_row0(it):
            c = wid + it * _NW
            r0 = pl.multiple_of(jnp.minimum(c * _DT_ROWS, last_r0), 64)
            return r0, c < n_chunks

        def in_start(it, p):
            r0, ok = chunk_row0(it)
            t0 = pl.multiple_of(r0 // 8, 8)

            @pl.when(ok)
            def _():
                pltpu.async_copy(w_hbm.at[pl.ds(t0, _DT_ROWS // 8)],
                                 vins[p], isems[p])

        def in_wait(it, p):
            r0, ok = chunk_row0(it)
            t0 = pl.multiple_of(r0 // 8, 8)

            @pl.when(ok)
            def _():
                pltpu.make_async_copy(w_hbm.at[pl.ds(t0, _DT_ROWS // 8)],
                                      vins[p], isems[p]).wait()

        def out_start(it, p):
            r0, ok = chunk_row0(it)
            q0 = pl.multiple_of(r0 // 4, 16)

            @pl.when(ok)
            def _():
                pltpu.async_copy(vouts[p],
                                 tbl_hbm.at[pl.ds(q0, _DT_Q)], osems[p])

        def out_wait(it, p):
            r0, ok = chunk_row0(it)
            q0 = pl.multiple_of(r0 // 4, 16)

            @pl.when(ok)
            def _():
                pltpu.make_async_copy(vouts[p],
                                      tbl_hbm.at[pl.ds(q0, _DT_Q)],
                                      osems[p]).wait()

        def strip(p):
            def group_body(i, carry):
                for rr in range(_L):
                    q = 4 * i + rr // 4
                    col = _D * (rr % 4)
                    t = 2 * i + rr // 8
                    s = rr % 8
                    vouts[p][q, pl.ds(col, _L)] = vins[p][t, s, pl.ds(0, _L)]
                    vouts[p][q, pl.ds(col + _L, _L)] = \
                        vins[p][t, s, pl.ds(_L, _L)]
                return carry
            lax.fori_loop(0, _DT_ROWS // _L, group_body, 0)

        in_start(0, 0)

        def pair_body(gp, carry):
            for sub in (0, 1):
                it = 2 * gp + sub
                in_start(it + 1, 1 - sub)
                in_wait(it, sub)

                @pl.when(it >= 2)
                def _():
                    out_wait(it - 2, sub)

                strip(sub)
                out_start(it, sub)
            return carry

        lax.fori_loop(0, n_iter // 2, pair_body, 0)
        it = n_iter - 1
        sub = it % 2
        in_wait(it, sub)
        out_wait(it - 2, sub)
        strip(sub)
        out_start(it, sub)
        out_wait(it - 1, 1 - sub)
        out_wait(it, sub)

    return detile


def kernel(x, weight):
    idx = x.reshape(-1).astype(jnp.int32)
    tbl = _make_detile_kernel(weight.shape[0])(weight.reshape(-1, 8, _D))
    out_fat = _make_sc_kernel()(idx, tbl)
    return out_fat.reshape(_B, _TP, _FAT)[:, :_T, :_D]
